# Initial kernel scaffold; baseline (speedup 1.0000x reference)
import jax
import jax.numpy as jnp
from jax.experimental import pallas as pl


def kernel(vertices):
    raise NotImplementedError("write your pallas kernel here")


# speedup vs baseline: 46.4681x; 46.4681x over previous
import jax
import jax.numpy as jnp
from jax.experimental import pallas as pl

_B = 4
_N = 4096
_K = 128
_R = 256


def _b16(x):
    u = jax.lax.bitcast_convert_type(x, jnp.int32)
    r = u + jnp.int32(0x7FFF) + (jax.lax.shift_right_logical(u, 16) & jnp.int32(1))
    r = r & jnp.int32(-65536)
    return jax.lax.bitcast_convert_type(r, jnp.float32)


def _row_cols(v_ref, vT_ref):
    vr = [v_ref[0, :, a:a + 1] for a in range(3)]
    vc = [vT_ref[0, a:a + 1, :] for a in range(3)]
    return vr, vc


def _d2_clamped(v_ref, vT_ref, vr, vc):
    sqr = (vr[0] * vr[0] + vr[2] * vr[2]) + vr[1] * vr[1]
    sqc = (vc[0] * vc[0] + vc[2] * vc[2]) + vc[1] * vc[1]
    mm = jax.lax.dot_general(
        v_ref[0].astype(jnp.bfloat16), vT_ref[0].astype(jnp.bfloat16),
        (((1,), (0,)), ((), ())), preferred_element_type=jnp.float32)
    d2 = (sqr - 2.0 * mm) + sqc
    return jnp.maximum(d2, 0.0)


def _radius2(d2c):
    bits = jax.lax.bitcast_convert_type(d2c, jnp.int32)
    lo0 = jnp.zeros((d2c.shape[0], 1), jnp.int32)
    hi0 = jnp.full((d2c.shape[0], 1), 0x7F7FFFFF, jnp.int32)

    def body(_, lh):
        lo, hi = lh
        mid = lo + (hi - lo) // 2
        cnt = jnp.sum((bits <= mid).astype(jnp.float32), axis=1, keepdims=True)
        ge = cnt >= float(_K)
        return jnp.where(ge, lo, mid + 1), jnp.where(ge, mid, hi)

    lo, hi = jax.lax.fori_loop(0, 31, body, (lo0, hi0))
    return jax.lax.bitcast_convert_type(hi, jnp.float32)


def _phase1(v_ref, vT_ref, nrm_ref, r2_ref):
    vr, vc = _row_cols(v_ref, vT_ref)
    d2c = _d2_clamped(v_ref, vT_ref, vr, vc)
    r2 = _radius2(d2c)
    radius = jnp.sqrt(r2)
    dist = jnp.sqrt(d2c)
    w = jnp.maximum(radius - dist, 0.0)
    mask = (d2c <= r2).astype(jnp.float32)
    wsum = jnp.sum(w, axis=1, keepdims=True)

    dd = [vc[a] - vr[a] for a in range(3)]
    wd16 = [_b16(w * dd[a]) for a in range(3)]
    d16 = [_b16(dd[a]) for a in range(3)]

    def mom(a, bb):
        return jnp.sum(wd16[a] * d16[bb], axis=1, keepdims=True) / wsum

    A = [[None] * 3 for _ in range(3)]
    for a in range(3):
        A[a][a] = mom(a, a)
        for c in range(a + 1, 3):
            A[a][c] = (mom(a, c) + mom(c, a)) * 0.5
            A[c][a] = A[a][c]

    V = [[jnp.full_like(wsum, 1.0 if a == c else 0.0) for c in range(3)]
         for a in range(3)]

    for _ in range(6):
        for (p, q) in ((0, 2), (1, 2), (0, 1)):
            app = A[p][p]; aqq = A[q][q]; apq = A[p][q]
            tau = (aqq - app) / (2.0 * apq)
            denom = jnp.abs(tau) + jnp.sqrt(1.0 + tau * tau)
            t = jnp.where(tau >= 0.0, 1.0, -1.0) / denom
            t = jnp.where(apq == 0.0, 0.0, t)
            c_ = 1.0 / jnp.sqrt(1.0 + t * t)
            s_ = t * c_
            r = 3 - p - q
            arp = A[r][p]; arq = A[r][q]
            new_app = c_ * (c_ * app - s_ * apq) - s_ * (c_ * apq - s_ * aqq)
            new_aqq = s_ * (s_ * app + c_ * apq) + c_ * (s_ * apq + c_ * aqq)
            new_apq = s_ * (c_ * app - s_ * apq) + c_ * (c_ * apq - s_ * aqq)
            A[p][p] = new_app; A[q][q] = new_aqq
            A[p][q] = new_apq; A[q][p] = new_apq
            A[r][p] = c_ * arp - s_ * arq
            A[p][r] = A[r][p]
            A[r][q] = s_ * arp + c_ * arq
            A[q][r] = A[r][q]
            for rr in range(3):
                vp = V[rr][p]; vq = V[rr][q]
                V[rr][p] = c_ * vp - s_ * vq
                V[rr][q] = s_ * vp + c_ * vq

    l0 = A[0][0]; l1 = A[1][1]; l2 = A[2][2]
    first0 = (l0 <= l1) & (l0 <= l2)
    first1 = l1 <= l2
    z = [jnp.where(first0, V[a][0], jnp.where(first1, V[a][1], V[a][2]))
         for a in range(3)]

    z16 = [_b16(z[a]) for a in range(3)]
    s = (d16[0] * z16[0] + d16[1] * z16[1]) + d16[2] * z16[2]
    ge_cnt = jnp.sum(mask * (s >= 0.0).astype(jnp.float32), axis=1,
                     keepdims=True)
    tot = jnp.sum(mask, axis=1, keepdims=True)
    sgn = jnp.where(ge_cnt >= tot - ge_cnt, 1.0, -1.0)
    for a in range(3):
        nrm_ref[0, :, a:a + 1] = z[a] * sgn
    r2_ref[0, :, 0:1] = r2


def _phase2(v_ref, vT_ref, nT_ref, r2_ref, out_ref):
    vr, vc = _row_cols(v_ref, vT_ref)
    d2c = _d2_clamped(v_ref, vT_ref, vr, vc)
    r2 = r2_ref[0, :, 0:1]
    mask = (d2c <= r2).astype(jnp.float32)
    avg = [jnp.sum(mask * nT_ref[0, a:a + 1, :], axis=1, keepdims=True)
           * (1.0 / _K) for a in range(3)]
    nrm = jnp.sqrt((avg[0] * avg[0] + avg[1] * avg[1]) + avg[2] * avg[2])
    for a in range(3):
        out_ref[0, :, a:a + 1] = avg[a] / nrm


def kernel(vertices):
    v = vertices
    vT = jnp.transpose(v, (0, 2, 1))
    grid = (_B, _N // _R)
    normals, r2 = pl.pallas_call(
        _phase1,
        grid=grid,
        in_specs=[
            pl.BlockSpec((1, _R, 3), lambda b, i: (b, i, 0)),
            pl.BlockSpec((1, 3, _N), lambda b, i: (b, 0, 0)),
        ],
        out_specs=[
            pl.BlockSpec((1, _R, 3), lambda b, i: (b, i, 0)),
            pl.BlockSpec((1, _R, 1), lambda b, i: (b, i, 0)),
        ],
        out_shape=[
            jax.ShapeDtypeStruct((_B, _N, 3), jnp.float32),
            jax.ShapeDtypeStruct((_B, _N, 1), jnp.float32),
        ],
    )(v, vT)
    nT = jnp.transpose(normals, (0, 2, 1))
    out = pl.pallas_call(
        _phase2,
        grid=grid,
        in_specs=[
            pl.BlockSpec((1, _R, 3), lambda b, i: (b, i, 0)),
            pl.BlockSpec((1, 3, _N), lambda b, i: (b, 0, 0)),
            pl.BlockSpec((1, 3, _N), lambda b, i: (b, 0, 0)),
            pl.BlockSpec((1, _R, 1), lambda b, i: (b, i, 0)),
        ],
        out_specs=pl.BlockSpec((1, _R, 3), lambda b, i: (b, i, 0)),
        out_shape=jax.ShapeDtypeStruct((_B, _N, 3), jnp.float32),
    )(v, vT, nT, r2)
    return out


# speedup vs baseline: 47.6867x; 1.0262x over previous
import jax
import jax.numpy as jnp
from jax.experimental import pallas as pl

_B = 4
_N = 4096
_K = 128
_R = 256


def _b16(x):
    p = x * 65537.0
    return p - (p - x)


def _row_cols(v_ref, vT_ref):
    vr = [v_ref[0, :, a:a + 1] for a in range(3)]
    vc = [vT_ref[0, a:a + 1, :] for a in range(3)]
    return vr, vc


def _d2_clamped(v_ref, vT_ref, vr, vc):
    sqr = (vr[0] * vr[0] + vr[2] * vr[2]) + vr[1] * vr[1]
    sqc = (vc[0] * vc[0] + vc[2] * vc[2]) + vc[1] * vc[1]
    mm = jax.lax.dot_general(
        v_ref[0].astype(jnp.bfloat16), vT_ref[0].astype(jnp.bfloat16),
        (((1,), (0,)), ((), ())), preferred_element_type=jnp.float32)
    d2 = (sqr - 2.0 * mm) + sqc
    return jnp.maximum(d2, 0.0)


def _radius2(d2c):
    bits = jax.lax.bitcast_convert_type(d2c, jnp.int32)
    lo0 = jnp.zeros((d2c.shape[0], 1), jnp.int32)
    hi0 = jnp.full((d2c.shape[0], 1), 0x7F7FFFFF, jnp.int32)

    def body(_, lh):
        lo, hi = lh
        mid = lo + (hi - lo) // 2
        cnt = jnp.sum((bits <= mid).astype(jnp.float32), axis=1, keepdims=True)
        ge = cnt >= float(_K)
        return jnp.where(ge, lo, mid + 1), jnp.where(ge, mid, hi)

    lo, hi = jax.lax.fori_loop(0, 31, body, (lo0, hi0))
    return jax.lax.bitcast_convert_type(hi, jnp.float32)


def _phase1(v_ref, vT_ref, nrm_ref, r2_ref):
    vr, vc = _row_cols(v_ref, vT_ref)
    d2c = _d2_clamped(v_ref, vT_ref, vr, vc)
    r2 = _radius2(d2c)
    radius = jnp.sqrt(r2)
    dist = jnp.sqrt(d2c)
    w = jnp.maximum(radius - dist, 0.0)
    mask = (d2c <= r2).astype(jnp.float32)
    wsum = jnp.sum(w, axis=1, keepdims=True)

    dd = [vc[a] - vr[a] for a in range(3)]
    wd16 = [_b16(w * dd[a]) for a in range(3)]
    d16 = [_b16(dd[a]) for a in range(3)]

    def mom(a, bb):
        return jnp.sum(wd16[a] * d16[bb], axis=1, keepdims=True) / wsum

    A = [[None] * 3 for _ in range(3)]
    for a in range(3):
        A[a][a] = mom(a, a)
        for c in range(a + 1, 3):
            A[a][c] = (mom(a, c) + mom(c, a)) * 0.5
            A[c][a] = A[a][c]

    V = [[jnp.full_like(wsum, 1.0 if a == c else 0.0) for c in range(3)]
         for a in range(3)]

    for _ in range(6):
        for (p, q) in ((0, 2), (1, 2), (0, 1)):
            app = A[p][p]; aqq = A[q][q]; apq = A[p][q]
            tau = (aqq - app) / (2.0 * apq)
            denom = jnp.abs(tau) + jnp.sqrt(1.0 + tau * tau)
            t = jnp.where(tau >= 0.0, 1.0, -1.0) / denom
            t = jnp.where(apq == 0.0, 0.0, t)
            c_ = 1.0 / jnp.sqrt(1.0 + t * t)
            s_ = t * c_
            r = 3 - p - q
            arp = A[r][p]; arq = A[r][q]
            new_app = c_ * (c_ * app - s_ * apq) - s_ * (c_ * apq - s_ * aqq)
            new_aqq = s_ * (s_ * app + c_ * apq) + c_ * (s_ * apq + c_ * aqq)
            new_apq = s_ * (c_ * app - s_ * apq) + c_ * (c_ * apq - s_ * aqq)
            A[p][p] = new_app; A[q][q] = new_aqq
            A[p][q] = new_apq; A[q][p] = new_apq
            A[r][p] = c_ * arp - s_ * arq
            A[p][r] = A[r][p]
            A[r][q] = s_ * arp + c_ * arq
            A[q][r] = A[r][q]
            for rr in range(3):
                vp = V[rr][p]; vq = V[rr][q]
                V[rr][p] = c_ * vp - s_ * vq
                V[rr][q] = s_ * vp + c_ * vq

    l0 = A[0][0]; l1 = A[1][1]; l2 = A[2][2]
    first0 = (l0 <= l1) & (l0 <= l2)
    first1 = l1 <= l2
    z = [jnp.where(first0, V[a][0], jnp.where(first1, V[a][1], V[a][2]))
         for a in range(3)]

    z16 = [_b16(z[a]) for a in range(3)]
    s = (d16[0] * z16[0] + d16[1] * z16[1]) + d16[2] * z16[2]
    ge_cnt = jnp.sum(mask * (s >= 0.0).astype(jnp.float32), axis=1,
                     keepdims=True)
    tot = jnp.sum(mask, axis=1, keepdims=True)
    sgn = jnp.where(ge_cnt >= tot - ge_cnt, 1.0, -1.0)
    for a in range(3):
        nrm_ref[0, :, a:a + 1] = z[a] * sgn
    r2_ref[0, :, 0:1] = r2


def _phase2(v_ref, vT_ref, nT_ref, r2_ref, out_ref):
    vr, vc = _row_cols(v_ref, vT_ref)
    d2c = _d2_clamped(v_ref, vT_ref, vr, vc)
    r2 = r2_ref[0, :, 0:1]
    mask = (d2c <= r2).astype(jnp.float32)
    avg = [jnp.sum(mask * nT_ref[0, a:a + 1, :], axis=1, keepdims=True)
           * (1.0 / _K) for a in range(3)]
    nrm = jnp.sqrt((avg[0] * avg[0] + avg[1] * avg[1]) + avg[2] * avg[2])
    for a in range(3):
        out_ref[0, :, a:a + 1] = avg[a] / nrm


def kernel(vertices):
    v = vertices
    vT = jnp.transpose(v, (0, 2, 1))
    grid = (_B, _N // _R)
    normals, r2 = pl.pallas_call(
        _phase1,
        grid=grid,
        in_specs=[
            pl.BlockSpec((1, _R, 3), lambda b, i: (b, i, 0)),
            pl.BlockSpec((1, 3, _N), lambda b, i: (b, 0, 0)),
        ],
        out_specs=[
            pl.BlockSpec((1, _R, 3), lambda b, i: (b, i, 0)),
            pl.BlockSpec((1, _R, 1), lambda b, i: (b, i, 0)),
        ],
        out_shape=[
            jax.ShapeDtypeStruct((_B, _N, 3), jnp.float32),
            jax.ShapeDtypeStruct((_B, _N, 1), jnp.float32),
        ],
    )(v, vT)
    nT = jnp.transpose(normals, (0, 2, 1))
    out = pl.pallas_call(
        _phase2,
        grid=grid,
        in_specs=[
            pl.BlockSpec((1, _R, 3), lambda b, i: (b, i, 0)),
            pl.BlockSpec((1, 3, _N), lambda b, i: (b, 0, 0)),
            pl.BlockSpec((1, 3, _N), lambda b, i: (b, 0, 0)),
            pl.BlockSpec((1, _R, 1), lambda b, i: (b, i, 0)),
        ],
        out_specs=pl.BlockSpec((1, _R, 3), lambda b, i: (b, i, 0)),
        out_shape=jax.ShapeDtypeStruct((_B, _N, 3), jnp.float32),
    )(v, vT, nT, r2)
    return out
